# baseline (speedup 1.0000x reference)
import jax
import jax.numpy as jnp
from jax import lax
from jax.experimental import pallas as pl
from jax.experimental.pallas import tpu as pltpu

_N = 20064
_ROWS = 160
_LANES = 128
_NPAD = _ROWS * _LANES
_CHUNKS = _ROWS // 8
_G = 64
_C = 21
_K = 64

_FG_THRESH = 0.7
_BG_HI = 0.5
_BG_LO = 0.1
_DOT = dict(preferred_element_type=jnp.float32,
            precision=lax.Precision.HIGHEST)


def _proposal_target_kernel(gt_smem, planes_ref, gtall_ref,
                            rois_ref, labels_ref, bbox_ref,
                            asg_ref, fg_ref, bg_ref,
                            rsel_ref, csel_ref, cmax_ref):
    f32 = jnp.float32
    i32 = jnp.int32

    row_i = lax.broadcasted_iota(i32, (8, _LANES), 0)
    lane_i = lax.broadcasted_iota(i32, (8, _LANES), 1)
    rel_i = row_i * _LANES + lane_i
    li = lax.broadcasted_iota(i32, (1, _LANES), 1)
    ri160 = lax.broadcasted_iota(i32, (1, _ROWS), 1)

    def iou_chunk(k, _):
        s = k * 8
        ax1 = planes_ref[0, pl.ds(s, 8), :]
        ay1 = planes_ref[1, pl.ds(s, 8), :]
        ax2 = planes_ref[2, pl.ds(s, 8), :]
        ay2 = planes_ref[3, pl.ds(s, 8), :]
        area_a = (ax2 - ax1 + 1.0) * (ay2 - ay1 + 1.0)
        maxv = jnp.full((8, _LANES), -1.0, f32)
        asg = jnp.zeros((8, _LANES), f32)

        def per_gt(g, carry):
            maxv, asg = carry
            bx1 = gt_smem[0, g]
            by1 = gt_smem[1, g]
            bx2 = gt_smem[2, g]
            by2 = gt_smem[3, g]
            area_b = (bx2 - bx1 + 1.0) * (by2 - by1 + 1.0)
            iw = jnp.maximum(
                jnp.minimum(ax2, bx2) - jnp.maximum(ax1, bx1) + 1.0, 0.0)
            ih = jnp.maximum(
                jnp.minimum(ay2, by2) - jnp.maximum(ay1, by1) + 1.0, 0.0)
            inter = iw * ih
            union = area_a + area_b - inter
            iou = inter / jnp.maximum(union, 1e-8)
            upd = iou > maxv
            asg = jnp.where(upd, g.astype(f32), asg)
            maxv = jnp.where(upd, iou, maxv)
            return maxv, asg

        maxv, asg = lax.fori_loop(0, _G, per_gt, (maxv, asg))

        flat = s * _LANES + rel_i
        valid = flat < _N
        fg = jnp.where(valid & (maxv >= _FG_THRESH), maxv, -1.0)
        fg = jnp.where(valid, fg, -2.0)
        bg = jnp.where(valid & (maxv < _BG_HI) & (maxv >= _BG_LO), maxv, -1.0)
        bg = jnp.where(valid, bg, -2.0)
        asg_ref[pl.ds(s, 8), :] = asg
        fg_ref[pl.ds(s, 8), :] = fg
        bg_ref[pl.ds(s, 8), :] = bg
        cmax_ref[k] = jnp.max(fg)
        cmax_ref[_CHUNKS + k] = jnp.max(bg)
        return 0

    lax.fori_loop(0, _CHUNKS, iou_chunk, 0)

    big = jnp.int32(1 << 30)

    def pick(ref, base, slot):
        best = cmax_ref[base]
        bk = jnp.int32(0)
        def scan_c(k, carry):
            best, bk = carry
            s = cmax_ref[base + k]
            upd = s > best
            return jnp.where(upd, s, best), jnp.where(upd, k, bk)
        best, bk = lax.fori_loop(1, _CHUNKS, scan_c, (best, bk))

        v = ref[pl.ds(bk * 8, 8), :]
        rel = jnp.min(jnp.where(v == best, rel_i, big))
        vn = jnp.where(rel_i == rel, -2.0, v)
        ref[pl.ds(bk * 8, 8), :] = vn
        cmax_ref[base + bk] = jnp.max(vn)
        r = bk * 8 + rel // _LANES
        c = rel % _LANES
        rsel_ref[pl.ds(slot, 1), :] = jnp.where(ri160 == r, 1.0, 0.0)
        csel_ref[pl.ds(slot, 1), :] = jnp.where(li == c, 1.0, 0.0)
        return 0

    def step(j, _):
        pick(fg_ref, 0, j)
        pick(bg_ref, _CHUNKS, _K + j)
        return 0

    lax.fori_loop(0, _K, step, 0)

    R = rsel_ref[...]
    C = csel_ref[...]
    dn = (((1,), (1,)), ((), ()))

    def gather_plane(p):
        t = lax.dot_general(C, p, dn, **_DOT)
        return jnp.sum(t * R, axis=1, keepdims=True)

    ex1 = gather_plane(planes_ref[0])
    ey1 = gather_plane(planes_ref[1])
    ex2 = gather_plane(planes_ref[2])
    ey2 = gather_plane(planes_ref[3])
    a = gather_plane(asg_ref[...])

    rois_out = jnp.where(li == 0, ex1,
                jnp.where(li == 1, ey1,
                 jnp.where(li == 2, ex2, ey2)))
    rois_ref[...] = rois_out

    gi64 = lax.broadcasted_iota(i32, (1, _G), 1)
    A = jnp.where(a[:_K].astype(i32) == gi64, 1.0, 0.0)
    gtg = lax.dot_general(A, gtall_ref[...],
                          (((1,), (0,)), ((), ())), **_DOT)

    def gext(lane):
        return jnp.sum(jnp.where(li == lane, gtg, 0.0),
                       axis=1, keepdims=True)
    gx1, gy1, gx2, gy2 = gext(24), gext(25), gext(26), gext(27)

    fx1, fy1, fx2, fy2 = ex1[:_K], ey1[:_K], ex2[:_K], ey2[:_K]
    ex_w = fx2 - fx1 + 1.0
    ex_h = fy2 - fy1 + 1.0
    ex_cx = fx1 + 0.5 * ex_w
    ex_cy = fy1 + 0.5 * ex_h
    gt_w = gx2 - gx1 + 1.0
    gt_h = gy2 - gy1 + 1.0
    gt_cx = gx1 + 0.5 * gt_w
    gt_cy = gy1 + 0.5 * gt_h
    dx = (gt_cx - ex_cx) / ex_w
    dy = (gt_cy - ex_cy) / ex_h
    dw = jnp.log(gt_w / ex_w)
    dh = jnp.log(gt_h / ex_h)

    bbox_fg = jnp.where(li == 0, dx,
               jnp.where(li == 1, dy,
                jnp.where(li == 2, dw, dh)))
    bbox_ref[pl.ds(0, _K), :] = bbox_fg
    bbox_ref[pl.ds(_K, _K), :] = jnp.zeros((_K, _LANES), f32)

    labels_ref[pl.ds(0, _K), :] = gtg
    labels_ref[pl.ds(_K, _K), :] = jnp.where(li == 0,
                                             jnp.ones((_K, _LANES), f32), 0.0)


def kernel(proposals, bounding_boxes, labels):
    f32 = jnp.float32
    p = jnp.concatenate([proposals[0], bounding_boxes[0]], axis=0)
    pp = jnp.pad(p, ((0, _NPAD - _N), (0, 0)))
    planes = pp.T.reshape(4, _ROWS, _LANES)
    gt = bounding_boxes[0]
    lab = labels[0]
    gtall = jnp.zeros((_G, _LANES), f32)
    gtall = gtall.at[:, :_C].set(lab)
    gtall = gtall.at[:, 24:28].set(gt)
    gt_smem = gt.T

    out_shape = [jax.ShapeDtypeStruct((2 * _K, _LANES), f32)] * 3
    rois, labels_out, bbox = pl.pallas_call(
        _proposal_target_kernel,
        out_shape=out_shape,
        in_specs=[
            pl.BlockSpec(memory_space=pltpu.SMEM),
            pl.BlockSpec(memory_space=pltpu.VMEM),
            pl.BlockSpec(memory_space=pltpu.VMEM),
        ],
        out_specs=[pl.BlockSpec(memory_space=pltpu.VMEM)] * 3,
        scratch_shapes=[
            pltpu.VMEM((_ROWS, _LANES), f32),
            pltpu.VMEM((_ROWS, _LANES), f32),
            pltpu.VMEM((_ROWS, _LANES), f32),
            pltpu.VMEM((2 * _K, _ROWS), f32),
            pltpu.VMEM((2 * _K, _LANES), f32),
            pltpu.SMEM((2 * _CHUNKS,), f32),
        ],
    )(gt_smem, planes, gtall)
    return (rois[None, :, :4], labels_out[None, :, :_C], bbox[None, :, :4])


# speedup vs baseline: 1.4680x; 1.4680x over previous
import jax
import jax.numpy as jnp
from jax import lax
from jax.experimental import pallas as pl
from jax.experimental.pallas import tpu as pltpu

_N = 20064
_ROWS = 160
_LANES = 128
_NPAD = _ROWS * _LANES
_CHUNKS = _ROWS // 8
_G = 64
_C = 21
_K = 64

_FG_THRESH = 0.7
_BG_HI = 0.5
_BG_LO = 0.1
_DOT = dict(preferred_element_type=jnp.float32,
            precision=lax.Precision.HIGHEST)


def _proposal_target_kernel(gt_smem, planes_ref, gtall_ref,
                            rois_ref, labels_ref, bbox_ref,
                            asg_ref, fg_ref, bg_ref,
                            rsel_ref, csel_ref):
    f32 = jnp.float32
    i32 = jnp.int32

    row_i = lax.broadcasted_iota(i32, (8, _LANES), 0)
    lane_i = lax.broadcasted_iota(i32, (8, _LANES), 1)
    rel_i = row_i * _LANES + lane_i
    li = lax.broadcasted_iota(i32, (1, _LANES), 1)
    ri160 = lax.broadcasted_iota(i32, (1, _ROWS), 1)

    def iou_chunk(k, _):
        s = k * 8
        ax1 = planes_ref[0, pl.ds(s, 8), :]
        ay1 = planes_ref[1, pl.ds(s, 8), :]
        ax2 = planes_ref[2, pl.ds(s, 8), :]
        ay2 = planes_ref[3, pl.ds(s, 8), :]
        area_a = (ax2 - ax1 + 1.0) * (ay2 - ay1 + 1.0)
        maxv = jnp.full((8, _LANES), -1.0, f32)
        asg = jnp.zeros((8, _LANES), f32)

        def per_gt(g, carry):
            maxv, asg = carry
            bx1 = gt_smem[0, g]
            by1 = gt_smem[1, g]
            bx2 = gt_smem[2, g]
            by2 = gt_smem[3, g]
            area_b = (bx2 - bx1 + 1.0) * (by2 - by1 + 1.0)
            iw = jnp.maximum(
                jnp.minimum(ax2, bx2) - jnp.maximum(ax1, bx1) + 1.0, 0.0)
            ih = jnp.maximum(
                jnp.minimum(ay2, by2) - jnp.maximum(ay1, by1) + 1.0, 0.0)
            inter = iw * ih
            union = area_a + area_b - inter
            iou = inter / jnp.maximum(union, 1e-8)
            upd = iou > maxv
            asg = jnp.where(upd, g.astype(f32), asg)
            maxv = jnp.where(upd, iou, maxv)
            return maxv, asg

        maxv, asg = lax.fori_loop(0, _G, per_gt, (maxv, asg))

        flat = s * _LANES + rel_i
        valid = flat < _N
        fg = jnp.where(valid & (maxv >= _FG_THRESH), maxv, -1.0)
        fg = jnp.where(valid, fg, -2.0)
        bg = jnp.where(valid & (maxv < _BG_HI) & (maxv >= _BG_LO), maxv, -1.0)
        bg = jnp.where(valid, bg, -2.0)
        asg_ref[pl.ds(s, 8), :] = asg
        fg_ref[pl.ds(s, 8), :] = fg
        bg_ref[pl.ds(s, 8), :] = bg
        return 0

    lax.fori_loop(0, _CHUNKS, iou_chunk, 0)

    big = jnp.int32(1 << 30)
    frow_i = lax.broadcasted_iota(i32, (_ROWS, _LANES), 0)
    flane_i = lax.broadcasted_iota(i32, (_ROWS, _LANES), 1)
    flat_all = frow_i * _LANES + flane_i

    def pick(ref, slot):
        v = ref[...]
        m = jnp.max(v)
        idx = jnp.min(jnp.where(v == m, flat_all, big))
        ref[...] = jnp.where(flat_all == idx, -2.0, v)
        r = idx // _LANES
        c = idx % _LANES
        rsel_ref[pl.ds(slot, 1), :] = jnp.where(ri160 == r, 1.0, 0.0)
        csel_ref[pl.ds(slot, 1), :] = jnp.where(li == c, 1.0, 0.0)
        return 0

    def step(j, _):
        pick(fg_ref, j)
        pick(bg_ref, _K + j)
        return 0

    lax.fori_loop(0, _K, step, 0)

    R = rsel_ref[...]
    C = csel_ref[...]
    dn = (((1,), (1,)), ((), ()))

    def gather_plane(p):
        t = lax.dot_general(C, p, dn, **_DOT)
        return jnp.sum(t * R, axis=1, keepdims=True)

    ex1 = gather_plane(planes_ref[0])
    ey1 = gather_plane(planes_ref[1])
    ex2 = gather_plane(planes_ref[2])
    ey2 = gather_plane(planes_ref[3])
    a = gather_plane(asg_ref[...])

    rois_out = jnp.where(li == 0, ex1,
                jnp.where(li == 1, ey1,
                 jnp.where(li == 2, ex2, ey2)))
    rois_ref[...] = rois_out

    gi64 = lax.broadcasted_iota(i32, (1, _G), 1)
    A = jnp.where(a[:_K].astype(i32) == gi64, 1.0, 0.0)
    gtg = lax.dot_general(A, gtall_ref[...],
                          (((1,), (0,)), ((), ())), **_DOT)

    def gext(lane):
        return jnp.sum(jnp.where(li == lane, gtg, 0.0),
                       axis=1, keepdims=True)
    gx1, gy1, gx2, gy2 = gext(24), gext(25), gext(26), gext(27)

    fx1, fy1, fx2, fy2 = ex1[:_K], ey1[:_K], ex2[:_K], ey2[:_K]
    ex_w = fx2 - fx1 + 1.0
    ex_h = fy2 - fy1 + 1.0
    ex_cx = fx1 + 0.5 * ex_w
    ex_cy = fy1 + 0.5 * ex_h
    gt_w = gx2 - gx1 + 1.0
    gt_h = gy2 - gy1 + 1.0
    gt_cx = gx1 + 0.5 * gt_w
    gt_cy = gy1 + 0.5 * gt_h
    dx = (gt_cx - ex_cx) / ex_w
    dy = (gt_cy - ex_cy) / ex_h
    dw = jnp.log(gt_w / ex_w)
    dh = jnp.log(gt_h / ex_h)

    bbox_fg = jnp.where(li == 0, dx,
               jnp.where(li == 1, dy,
                jnp.where(li == 2, dw, dh)))
    bbox_ref[pl.ds(0, _K), :] = bbox_fg
    bbox_ref[pl.ds(_K, _K), :] = jnp.zeros((_K, _LANES), f32)

    labels_ref[pl.ds(0, _K), :] = gtg
    labels_ref[pl.ds(_K, _K), :] = jnp.where(li == 0,
                                             jnp.ones((_K, _LANES), f32), 0.0)


def kernel(proposals, bounding_boxes, labels):
    f32 = jnp.float32
    p = jnp.concatenate([proposals[0], bounding_boxes[0]], axis=0)
    pp = jnp.pad(p, ((0, _NPAD - _N), (0, 0)))
    planes = pp.T.reshape(4, _ROWS, _LANES)
    gt = bounding_boxes[0]
    lab = labels[0]
    gtall = jnp.zeros((_G, _LANES), f32)
    gtall = gtall.at[:, :_C].set(lab)
    gtall = gtall.at[:, 24:28].set(gt)
    gt_smem = gt.T

    out_shape = [jax.ShapeDtypeStruct((2 * _K, _LANES), f32)] * 3
    rois, labels_out, bbox = pl.pallas_call(
        _proposal_target_kernel,
        out_shape=out_shape,
        in_specs=[
            pl.BlockSpec(memory_space=pltpu.SMEM),
            pl.BlockSpec(memory_space=pltpu.VMEM),
            pl.BlockSpec(memory_space=pltpu.VMEM),
        ],
        out_specs=[pl.BlockSpec(memory_space=pltpu.VMEM)] * 3,
        scratch_shapes=[
            pltpu.VMEM((_ROWS, _LANES), f32),
            pltpu.VMEM((_ROWS, _LANES), f32),
            pltpu.VMEM((_ROWS, _LANES), f32),
            pltpu.VMEM((2 * _K, _ROWS), f32),
            pltpu.VMEM((2 * _K, _LANES), f32),
        ],
    )(gt_smem, planes, gtall)
    return (rois[None, :, :4], labels_out[None, :, :_C], bbox[None, :, :4])


# speedup vs baseline: 2.7144x; 1.8490x over previous
import jax
import jax.numpy as jnp
from jax import lax
from jax.experimental import pallas as pl
from jax.experimental.pallas import tpu as pltpu

_N = 20064
_ROWS = 160
_LANES = 128
_NPAD = _ROWS * _LANES
_CHUNKS = _ROWS // 8
_G = 64
_C = 21
_K = 64

_FG_THRESH = 0.7
_BG_HI = 0.5
_BG_LO = 0.1
_DOT = dict(preferred_element_type=jnp.float32,
            precision=lax.Precision.HIGHEST)


def _proposal_target_kernel(gt_smem, planes_ref, gtall_ref,
                            rois_ref, labels_ref, bbox_ref,
                            asg_ref, fg_ref, bg_ref, idxsel_ref):
    f32 = jnp.float32
    i32 = jnp.int32

    row_i = lax.broadcasted_iota(i32, (8, _LANES), 0)
    lane_i = lax.broadcasted_iota(i32, (8, _LANES), 1)
    rel_i = row_i * _LANES + lane_i
    li = lax.broadcasted_iota(i32, (1, _LANES), 1)
    ri160 = lax.broadcasted_iota(i32, (1, _ROWS), 1)

    def iou_chunk(k, _):
        s = k * 8
        ax1 = planes_ref[0, pl.ds(s, 8), :]
        ay1 = planes_ref[1, pl.ds(s, 8), :]
        ax2 = planes_ref[2, pl.ds(s, 8), :]
        ay2 = planes_ref[3, pl.ds(s, 8), :]
        area_a = (ax2 - ax1 + 1.0) * (ay2 - ay1 + 1.0)
        maxv = jnp.full((8, _LANES), -1.0, f32)
        asg = jnp.zeros((8, _LANES), f32)

        for g in range(_G):
            bx1 = gt_smem[0, g]
            by1 = gt_smem[1, g]
            bx2 = gt_smem[2, g]
            by2 = gt_smem[3, g]
            area_b = (bx2 - bx1 + 1.0) * (by2 - by1 + 1.0)
            iw = jnp.maximum(
                jnp.minimum(ax2, bx2) - jnp.maximum(ax1, bx1) + 1.0, 0.0)
            ih = jnp.maximum(
                jnp.minimum(ay2, by2) - jnp.maximum(ay1, by1) + 1.0, 0.0)
            inter = iw * ih
            union = area_a + area_b - inter
            iou = inter / jnp.maximum(union, 1e-8)
            upd = iou > maxv
            asg = jnp.where(upd, float(g), asg)
            maxv = jnp.where(upd, iou, maxv)

        flat = s * _LANES + rel_i
        valid = flat < _N
        fg = jnp.where(valid & (maxv >= _FG_THRESH), maxv, -1.0)
        fg = jnp.where(valid, fg, -2.0)
        bg = jnp.where(valid & (maxv < _BG_HI) & (maxv >= _BG_LO), maxv, -1.0)
        bg = jnp.where(valid, bg, -2.0)
        asg_ref[pl.ds(s, 8), :] = asg
        fg_ref[pl.ds(s, 8), :] = fg
        bg_ref[pl.ds(s, 8), :] = bg
        return 0

    lax.fori_loop(0, _CHUNKS, iou_chunk, 0)

    big = jnp.int32(1 << 30)
    frow_i = lax.broadcasted_iota(i32, (_ROWS, _LANES), 0)
    flane_i = lax.broadcasted_iota(i32, (_ROWS, _LANES), 1)
    flat_all = frow_i * _LANES + flane_i

    def pick(ref, slot, m):
        v = ref[...]
        idx = jnp.min(jnp.where(v == m, flat_all, big),
                      axis=(0, 1), keepdims=True)
        vn = jnp.where(flat_all == idx, -2.0, v)
        ref[...] = vn
        idxsel_ref[pl.ds(slot, 1), :] = idx
        return jnp.max(vn, axis=(0, 1), keepdims=True)

    mfg = jnp.max(fg_ref[...], axis=(0, 1), keepdims=True)
    mbg = jnp.max(bg_ref[...], axis=(0, 1), keepdims=True)
    for j in range(_K):
        mfg = pick(fg_ref, j, mfg)
        mbg = pick(bg_ref, _K + j, mbg)

    idxs = idxsel_ref[...]
    R = jnp.where(idxs // _LANES == ri160, 1.0, 0.0)
    C = jnp.where(idxs % _LANES == li, 1.0, 0.0)
    dn = (((1,), (1,)), ((), ()))

    def gather_plane(p):
        t = lax.dot_general(C, p, dn, **_DOT)
        return jnp.sum(t * R, axis=1, keepdims=True)

    ex1 = gather_plane(planes_ref[0])
    ey1 = gather_plane(planes_ref[1])
    ex2 = gather_plane(planes_ref[2])
    ey2 = gather_plane(planes_ref[3])
    a = gather_plane(asg_ref[...])

    rois_out = jnp.where(li == 0, ex1,
                jnp.where(li == 1, ey1,
                 jnp.where(li == 2, ex2, ey2)))
    rois_ref[...] = rois_out

    gi64 = lax.broadcasted_iota(i32, (1, _G), 1)
    A = jnp.where(a[:_K].astype(i32) == gi64, 1.0, 0.0)
    gtg = lax.dot_general(A, gtall_ref[...],
                          (((1,), (0,)), ((), ())), **_DOT)

    def gext(lane):
        return jnp.sum(jnp.where(li == lane, gtg, 0.0),
                       axis=1, keepdims=True)
    gx1, gy1, gx2, gy2 = gext(24), gext(25), gext(26), gext(27)

    fx1, fy1, fx2, fy2 = ex1[:_K], ey1[:_K], ex2[:_K], ey2[:_K]
    ex_w = fx2 - fx1 + 1.0
    ex_h = fy2 - fy1 + 1.0
    ex_cx = fx1 + 0.5 * ex_w
    ex_cy = fy1 + 0.5 * ex_h
    gt_w = gx2 - gx1 + 1.0
    gt_h = gy2 - gy1 + 1.0
    gt_cx = gx1 + 0.5 * gt_w
    gt_cy = gy1 + 0.5 * gt_h
    dx = (gt_cx - ex_cx) / ex_w
    dy = (gt_cy - ex_cy) / ex_h
    dw = jnp.log(gt_w / ex_w)
    dh = jnp.log(gt_h / ex_h)

    bbox_fg = jnp.where(li == 0, dx,
               jnp.where(li == 1, dy,
                jnp.where(li == 2, dw, dh)))
    bbox_ref[pl.ds(0, _K), :] = bbox_fg
    bbox_ref[pl.ds(_K, _K), :] = jnp.zeros((_K, _LANES), f32)

    labels_ref[pl.ds(0, _K), :] = gtg
    labels_ref[pl.ds(_K, _K), :] = jnp.where(li == 0,
                                             jnp.ones((_K, _LANES), f32), 0.0)


def kernel(proposals, bounding_boxes, labels):
    f32 = jnp.float32
    p = jnp.concatenate([proposals[0], bounding_boxes[0]], axis=0)
    pp = jnp.pad(p, ((0, _NPAD - _N), (0, 0)))
    planes = pp.T.reshape(4, _ROWS, _LANES)
    gt = bounding_boxes[0]
    lab = labels[0]
    gtall = jnp.zeros((_G, _LANES), f32)
    gtall = gtall.at[:, :_C].set(lab)
    gtall = gtall.at[:, 24:28].set(gt)
    gt_smem = gt.T

    out_shape = [jax.ShapeDtypeStruct((2 * _K, _LANES), f32)] * 3
    rois, labels_out, bbox = pl.pallas_call(
        _proposal_target_kernel,
        out_shape=out_shape,
        in_specs=[
            pl.BlockSpec(memory_space=pltpu.SMEM),
            pl.BlockSpec(memory_space=pltpu.VMEM),
            pl.BlockSpec(memory_space=pltpu.VMEM),
        ],
        out_specs=[pl.BlockSpec(memory_space=pltpu.VMEM)] * 3,
        scratch_shapes=[
            pltpu.VMEM((_ROWS, _LANES), f32),
            pltpu.VMEM((_ROWS, _LANES), f32),
            pltpu.VMEM((_ROWS, _LANES), f32),
            pltpu.VMEM((2 * _K, 1), jnp.int32),
        ],
    )(gt_smem, planes, gtall)
    return (rois[None, :, :4], labels_out[None, :, :_C], bbox[None, :, :4])


# speedup vs baseline: 3.1764x; 1.1702x over previous
import jax
import jax.numpy as jnp
from jax import lax
from jax.experimental import pallas as pl
from jax.experimental.pallas import tpu as pltpu

_N = 20064
_ROWS = 160
_LANES = 128
_NPAD = _ROWS * _LANES
_CHUNKS = _ROWS // 8
_G = 64
_C = 21
_K = 64

_FG_THRESH = 0.7
_BG_HI = 0.5
_BG_LO = 0.1
_DOT = dict(preferred_element_type=jnp.float32,
            precision=lax.Precision.HIGHEST)


def _proposal_target_kernel(gt_smem, planes_ref, gtall_ref,
                            rois_ref, labels_ref, bbox_ref,
                            asg_ref, fg_ref, bg_ref, idxsel_ref):
    f32 = jnp.float32
    i32 = jnp.int32

    row_i = lax.broadcasted_iota(i32, (8, _LANES), 0)
    lane_i = lax.broadcasted_iota(i32, (8, _LANES), 1)
    rel_i = row_i * _LANES + lane_i
    li = lax.broadcasted_iota(i32, (1, _LANES), 1)
    ri160 = lax.broadcasted_iota(i32, (1, _ROWS), 1)

    def iou_chunk(k, _):
        s = k * 8
        ax1 = planes_ref[0, pl.ds(s, 8), :]
        ay1 = planes_ref[1, pl.ds(s, 8), :]
        ax2 = planes_ref[2, pl.ds(s, 8), :]
        ay2 = planes_ref[3, pl.ds(s, 8), :]
        area_a = (ax2 - ax1 + 1.0) * (ay2 - ay1 + 1.0)
        maxv = jnp.full((8, _LANES), -1.0, f32)
        asg = jnp.zeros((8, _LANES), f32)

        for g in range(_G):
            bx1 = gt_smem[0, g]
            by1 = gt_smem[1, g]
            bx2 = gt_smem[2, g]
            by2 = gt_smem[3, g]
            area_b = (bx2 - bx1 + 1.0) * (by2 - by1 + 1.0)
            iw = jnp.maximum(
                jnp.minimum(ax2, bx2) - jnp.maximum(ax1, bx1) + 1.0, 0.0)
            ih = jnp.maximum(
                jnp.minimum(ay2, by2) - jnp.maximum(ay1, by1) + 1.0, 0.0)
            inter = iw * ih
            union = area_a + area_b - inter
            iou = inter / jnp.maximum(union, 1e-8)
            upd = iou > maxv
            asg = jnp.where(upd, float(g), asg)
            maxv = jnp.where(upd, iou, maxv)

        flat = s * _LANES + rel_i
        valid = flat < _N
        fg = jnp.where(valid & (maxv >= _FG_THRESH), maxv, -1.0)
        fg = jnp.where(valid, fg, -2.0)
        bg = jnp.where(valid & (maxv < _BG_HI) & (maxv >= _BG_LO), maxv, -1.0)
        bg = jnp.where(valid, bg, -2.0)
        asg_ref[pl.ds(s, 8), :] = asg
        fg_ref[pl.ds(s, 8), :] = fg
        bg_ref[pl.ds(s, 8), :] = bg
        return 0

    lax.fori_loop(0, _CHUNKS, iou_chunk, 0)

    big = jnp.int32(1 << 30)
    frow_i = lax.broadcasted_iota(i32, (_ROWS, _LANES), 0)
    flane_i = lax.broadcasted_iota(i32, (_ROWS, _LANES), 1)
    flat_all = frow_i * _LANES + flane_i

    def pick(ref, slot):
        v = ref[...]
        colmax = jnp.max(v, axis=0, keepdims=True)
        colrow = jnp.min(jnp.where(v == colmax, frow_i, big),
                         axis=0, keepdims=True)
        m = jnp.max(colmax, axis=1, keepdims=True)
        idx = jnp.min(jnp.where(colmax == m, colrow * _LANES + li, big),
                      axis=1, keepdims=True)
        ref[...] = jnp.where(flat_all == idx, -2.0, v)
        idxsel_ref[pl.ds(slot, 1), :] = idx

    for j in range(_K):
        pick(fg_ref, j)
        pick(bg_ref, _K + j)

    idxs = idxsel_ref[...]
    R = jnp.where(idxs // _LANES == ri160, 1.0, 0.0)
    C = jnp.where(idxs % _LANES == li, 1.0, 0.0)
    dn = (((1,), (1,)), ((), ()))

    def gather_plane(p):
        t = lax.dot_general(C, p, dn, **_DOT)
        return jnp.sum(t * R, axis=1, keepdims=True)

    ex1 = gather_plane(planes_ref[0])
    ey1 = gather_plane(planes_ref[1])
    ex2 = gather_plane(planes_ref[2])
    ey2 = gather_plane(planes_ref[3])
    a = gather_plane(asg_ref[...])

    rois_out = jnp.where(li == 0, ex1,
                jnp.where(li == 1, ey1,
                 jnp.where(li == 2, ex2, ey2)))
    rois_ref[...] = rois_out

    gi64 = lax.broadcasted_iota(i32, (1, _G), 1)
    A = jnp.where(a[:_K].astype(i32) == gi64, 1.0, 0.0)
    gtg = lax.dot_general(A, gtall_ref[...],
                          (((1,), (0,)), ((), ())), **_DOT)

    def gext(lane):
        return jnp.sum(jnp.where(li == lane, gtg, 0.0),
                       axis=1, keepdims=True)
    gx1, gy1, gx2, gy2 = gext(24), gext(25), gext(26), gext(27)

    fx1, fy1, fx2, fy2 = ex1[:_K], ey1[:_K], ex2[:_K], ey2[:_K]
    ex_w = fx2 - fx1 + 1.0
    ex_h = fy2 - fy1 + 1.0
    ex_cx = fx1 + 0.5 * ex_w
    ex_cy = fy1 + 0.5 * ex_h
    gt_w = gx2 - gx1 + 1.0
    gt_h = gy2 - gy1 + 1.0
    gt_cx = gx1 + 0.5 * gt_w
    gt_cy = gy1 + 0.5 * gt_h
    dx = (gt_cx - ex_cx) / ex_w
    dy = (gt_cy - ex_cy) / ex_h
    dw = jnp.log(gt_w / ex_w)
    dh = jnp.log(gt_h / ex_h)

    bbox_fg = jnp.where(li == 0, dx,
               jnp.where(li == 1, dy,
                jnp.where(li == 2, dw, dh)))
    bbox_ref[pl.ds(0, _K), :] = bbox_fg
    bbox_ref[pl.ds(_K, _K), :] = jnp.zeros((_K, _LANES), f32)

    labels_ref[pl.ds(0, _K), :] = gtg
    labels_ref[pl.ds(_K, _K), :] = jnp.where(li == 0,
                                             jnp.ones((_K, _LANES), f32), 0.0)


def kernel(proposals, bounding_boxes, labels):
    f32 = jnp.float32
    p = jnp.concatenate([proposals[0], bounding_boxes[0]], axis=0)
    pp = jnp.pad(p, ((0, _NPAD - _N), (0, 0)))
    planes = pp.T.reshape(4, _ROWS, _LANES)
    gt = bounding_boxes[0]
    lab = labels[0]
    gtall = jnp.zeros((_G, _LANES), f32)
    gtall = gtall.at[:, :_C].set(lab)
    gtall = gtall.at[:, 24:28].set(gt)
    gt_smem = gt.T

    out_shape = [jax.ShapeDtypeStruct((2 * _K, _LANES), f32)] * 3
    rois, labels_out, bbox = pl.pallas_call(
        _proposal_target_kernel,
        out_shape=out_shape,
        in_specs=[
            pl.BlockSpec(memory_space=pltpu.SMEM),
            pl.BlockSpec(memory_space=pltpu.VMEM),
            pl.BlockSpec(memory_space=pltpu.VMEM),
        ],
        out_specs=[pl.BlockSpec(memory_space=pltpu.VMEM)] * 3,
        scratch_shapes=[
            pltpu.VMEM((_ROWS, _LANES), f32),
            pltpu.VMEM((_ROWS, _LANES), f32),
            pltpu.VMEM((_ROWS, _LANES), f32),
            pltpu.VMEM((2 * _K, 1), jnp.int32),
        ],
    )(gt_smem, planes, gtall)
    return (rois[None, :, :4], labels_out[None, :, :_C], bbox[None, :, :4])


# speedup vs baseline: 3.4381x; 1.0824x over previous
import jax
import jax.numpy as jnp
from jax import lax
from jax.experimental import pallas as pl
from jax.experimental.pallas import tpu as pltpu

_N = 20064
_ROWS = 160
_LANES = 128
_NPAD = _ROWS * _LANES
_CHUNKS = _ROWS // 8
_G = 64
_C = 21
_K = 64

_FG_THRESH = 0.7
_BG_HI = 0.5
_BG_LO = 0.1
_DOT = dict(preferred_element_type=jnp.float32,
            precision=lax.Precision.HIGHEST)


def _proposal_target_kernel(gt_smem, planes_ref, gt_ref, lab_ref,
                            rois_ref, labels_ref, bbox_ref,
                            asg_ref, fg_ref, bg_ref, idxsel_ref):
    f32 = jnp.float32
    i32 = jnp.int32

    row_i = lax.broadcasted_iota(i32, (8, _LANES), 0)
    lane_i = lax.broadcasted_iota(i32, (8, _LANES), 1)
    rel_i = row_i * _LANES + lane_i
    li = lax.broadcasted_iota(i32, (1, _LANES), 1)
    li4 = lax.broadcasted_iota(i32, (1, 4), 1)
    li21 = lax.broadcasted_iota(i32, (1, _C), 1)
    ri160 = lax.broadcasted_iota(i32, (1, _ROWS), 1)
    fr_col = lax.broadcasted_iota(i32, (_ROWS, 1), 0)

    def iou_chunk(k, _):
        s = k * 8
        ax1 = planes_ref[0, pl.ds(s, 8), :]
        ay1 = planes_ref[1, pl.ds(s, 8), :]
        ax2 = planes_ref[2, pl.ds(s, 8), :]
        ay2 = planes_ref[3, pl.ds(s, 8), :]
        area_a = (ax2 - ax1 + 1.0) * (ay2 - ay1 + 1.0)
        maxv = jnp.full((8, _LANES), -1.0, f32)
        asg = jnp.zeros((8, _LANES), f32)

        for g in range(_G):
            bx1 = gt_smem[g, 0]
            by1 = gt_smem[g, 1]
            bx2 = gt_smem[g, 2]
            by2 = gt_smem[g, 3]
            area_b = (bx2 - bx1 + 1.0) * (by2 - by1 + 1.0)
            iw = jnp.maximum(
                jnp.minimum(ax2, bx2) - jnp.maximum(ax1, bx1) + 1.0, 0.0)
            ih = jnp.maximum(
                jnp.minimum(ay2, by2) - jnp.maximum(ay1, by1) + 1.0, 0.0)
            inter = iw * ih
            union = area_a + area_b - inter
            iou = inter / jnp.maximum(union, 1e-8)
            upd = iou > maxv
            asg = jnp.where(upd, float(g), asg)
            maxv = jnp.where(upd, iou, maxv)

        flat = s * _LANES + rel_i
        valid = flat < _N
        fg = jnp.where(valid & (maxv >= _FG_THRESH), maxv, -1.0)
        fg = jnp.where(valid, fg, -2.0)
        bg = jnp.where(valid & (maxv < _BG_HI) & (maxv >= _BG_LO), maxv, -1.0)
        bg = jnp.where(valid, bg, -2.0)
        asg_ref[pl.ds(s, 8), :] = asg
        fg_ref[pl.ds(s, 8), :] = fg
        bg_ref[pl.ds(s, 8), :] = bg
        return 0

    lax.fori_loop(0, _CHUNKS, iou_chunk, 0)

    big = jnp.int32(1 << 30)

    def pick(ref, slot):
        v = ref[...]
        colmax = jnp.max(v, axis=0, keepdims=True)
        colrow = jnp.min(jnp.where(v == colmax, fr_col, big),
                         axis=0, keepdims=True)
        m = jnp.max(colmax, axis=1, keepdims=True)
        idx = jnp.min(jnp.where(colmax == m, colrow * _LANES + li, big),
                      axis=1, keepdims=True)
        hit = (fr_col == idx // _LANES) & (li == idx % _LANES)
        ref[...] = jnp.where(hit, -2.0, v)
        idxsel_ref[pl.ds(slot, 1), :] = idx

    for j in range(_K):
        pick(fg_ref, j)
        pick(bg_ref, _K + j)

    idxs = idxsel_ref[...]
    R = jnp.where(idxs // _LANES == ri160, 1.0, 0.0)
    C = jnp.where(idxs % _LANES == li, 1.0, 0.0)
    dn = (((1,), (1,)), ((), ()))

    def gather_plane(p):
        t = lax.dot_general(C, p, dn, **_DOT)
        return jnp.sum(t * R, axis=1, keepdims=True)

    ex1 = gather_plane(planes_ref[0])
    ey1 = gather_plane(planes_ref[1])
    ex2 = gather_plane(planes_ref[2])
    ey2 = gather_plane(planes_ref[3])
    a = gather_plane(asg_ref[...])

    rois_ref[...] = jnp.where(li4 == 0, ex1,
                     jnp.where(li4 == 1, ey1,
                      jnp.where(li4 == 2, ex2, ey2)))

    gi64 = lax.broadcasted_iota(i32, (1, _G), 1)
    A = jnp.where(a[:_K].astype(i32) == gi64, 1.0, 0.0)
    dnr = (((1,), (0,)), ((), ()))
    glab = lax.dot_general(A, lab_ref[...], dnr, **_DOT)
    gbox = lax.dot_general(A, gt_ref[...], dnr, **_DOT)

    def gext(lane):
        return jnp.sum(jnp.where(li4 == lane, gbox, 0.0),
                       axis=1, keepdims=True)
    gx1, gy1, gx2, gy2 = gext(0), gext(1), gext(2), gext(3)

    fx1, fy1, fx2, fy2 = ex1[:_K], ey1[:_K], ex2[:_K], ey2[:_K]
    ex_w = fx2 - fx1 + 1.0
    ex_h = fy2 - fy1 + 1.0
    ex_cx = fx1 + 0.5 * ex_w
    ex_cy = fy1 + 0.5 * ex_h
    gt_w = gx2 - gx1 + 1.0
    gt_h = gy2 - gy1 + 1.0
    gt_cx = gx1 + 0.5 * gt_w
    gt_cy = gy1 + 0.5 * gt_h
    dx = (gt_cx - ex_cx) / ex_w
    dy = (gt_cy - ex_cy) / ex_h
    dw = jnp.log(gt_w / ex_w)
    dh = jnp.log(gt_h / ex_h)

    bbox_ref[pl.ds(0, _K), :] = jnp.where(li4 == 0, dx,
                                 jnp.where(li4 == 1, dy,
                                  jnp.where(li4 == 2, dw, dh)))
    bbox_ref[pl.ds(_K, _K), :] = jnp.zeros((_K, 4), f32)

    labels_ref[pl.ds(0, _K), :] = glab
    labels_ref[pl.ds(_K, _K), :] = jnp.where(li21 == 0,
                                             jnp.ones((_K, _C), f32), 0.0)


def kernel(proposals, bounding_boxes, labels):
    f32 = jnp.float32
    p = jnp.concatenate([proposals[0], bounding_boxes[0]], axis=0)
    pp = jnp.pad(p, ((0, _NPAD - _N), (0, 0)))
    planes = pp.T.reshape(4, _ROWS, _LANES)
    gt = bounding_boxes[0]
    lab = labels[0]

    out_shape = [jax.ShapeDtypeStruct((2 * _K, 4), f32),
                 jax.ShapeDtypeStruct((2 * _K, _C), f32),
                 jax.ShapeDtypeStruct((2 * _K, 4), f32)]
    rois, labels_out, bbox = pl.pallas_call(
        _proposal_target_kernel,
        out_shape=out_shape,
        in_specs=[
            pl.BlockSpec(memory_space=pltpu.SMEM),
            pl.BlockSpec(memory_space=pltpu.VMEM),
            pl.BlockSpec(memory_space=pltpu.VMEM),
            pl.BlockSpec(memory_space=pltpu.VMEM),
        ],
        out_specs=[pl.BlockSpec(memory_space=pltpu.VMEM)] * 3,
        scratch_shapes=[
            pltpu.VMEM((_ROWS, _LANES), f32),
            pltpu.VMEM((_ROWS, _LANES), f32),
            pltpu.VMEM((_ROWS, _LANES), f32),
            pltpu.VMEM((2 * _K, 1), jnp.int32),
        ],
    )(gt, planes, gt, lab)
    return (rois[None], labels_out[None], bbox[None])


# speedup vs baseline: 3.4445x; 1.0018x over previous
import jax
import jax.numpy as jnp
from jax import lax
from jax.experimental import pallas as pl
from jax.experimental.pallas import tpu as pltpu

_N = 20064
_ROWS = 160
_LANES = 128
_NPAD = _ROWS * _LANES
_CHUNKS = _ROWS // 8
_G = 64
_C = 21
_K = 64

_FG_THRESH = 0.7
_BG_HI = 0.5
_BG_LO = 0.1
_DOT = dict(preferred_element_type=jnp.float32,
            precision=lax.Precision.HIGHEST)


def _proposal_target_kernel(gt_smem, planes_ref, gt_ref, lab_ref,
                            rois_ref, labels_ref, bbox_ref,
                            asg_ref, fg_ref, bg_ref, idxsel_ref):
    f32 = jnp.float32
    i32 = jnp.int32

    row_i = lax.broadcasted_iota(i32, (8, _LANES), 0)
    lane_i = lax.broadcasted_iota(i32, (8, _LANES), 1)
    rel_i = row_i * _LANES + lane_i
    li = lax.broadcasted_iota(i32, (1, _LANES), 1)
    li4 = lax.broadcasted_iota(i32, (1, 4), 1)
    li21 = lax.broadcasted_iota(i32, (1, _C), 1)
    ri160 = lax.broadcasted_iota(i32, (1, _ROWS), 1)
    fr_col = lax.broadcasted_iota(i32, (_ROWS, 1), 0)

    def iou_chunk(k, _):
        s = k * 8
        ax1 = planes_ref[0, pl.ds(s, 8), :]
        ay1 = planes_ref[1, pl.ds(s, 8), :]
        ax2 = planes_ref[2, pl.ds(s, 8), :]
        ay2 = planes_ref[3, pl.ds(s, 8), :]
        area_a = (ax2 - ax1 + 1.0) * (ay2 - ay1 + 1.0)
        maxv = jnp.full((8, _LANES), -1.0, f32)
        asg = jnp.zeros((8, _LANES), f32)

        for g in range(_G):
            bx1 = gt_smem[g, 0]
            by1 = gt_smem[g, 1]
            bx2 = gt_smem[g, 2]
            by2 = gt_smem[g, 3]
            area_b = (bx2 - bx1 + 1.0) * (by2 - by1 + 1.0)
            iw = jnp.maximum(
                jnp.minimum(ax2, bx2) - jnp.maximum(ax1, bx1) + 1.0, 0.0)
            ih = jnp.maximum(
                jnp.minimum(ay2, by2) - jnp.maximum(ay1, by1) + 1.0, 0.0)
            inter = iw * ih
            union = area_a + area_b - inter
            iou = inter / jnp.maximum(union, 1e-8)
            upd = iou > maxv
            asg = jnp.where(upd, float(g), asg)
            maxv = jnp.where(upd, iou, maxv)

        flat = s * _LANES + rel_i
        valid = flat < _N
        fg = jnp.where(valid & (maxv >= _FG_THRESH), maxv, -1.0)
        fg = jnp.where(valid, fg, -2.0)
        bg = jnp.where(valid & (maxv < _BG_HI) & (maxv >= _BG_LO), maxv, -1.0)
        bg = jnp.where(valid, bg, -2.0)
        asg_ref[pl.ds(s, 8), :] = asg
        fg_ref[pl.ds(s, 8), :] = fg
        bg_ref[pl.ds(s, 8), :] = bg
        return 0

    lax.fori_loop(0, _CHUNKS, iou_chunk, 0)

    big = jnp.int32(1 << 30)

    def pick(v, slot):
        colmax = jnp.max(v, axis=0, keepdims=True)
        colrow = jnp.min(jnp.where(v == colmax, fr_col, big),
                         axis=0, keepdims=True)
        m = jnp.max(colmax, axis=1, keepdims=True)
        idx = jnp.min(jnp.where(colmax == m, colrow * _LANES + li, big),
                      axis=1, keepdims=True)
        hit = (fr_col == idx // _LANES) & (li == idx % _LANES)
        idxsel_ref[pl.ds(slot, 1), :] = idx
        return jnp.where(hit, -2.0, v)

    fgv = fg_ref[...]
    bgv = bg_ref[...]
    for j in range(_K):
        fgv = pick(fgv, j)
        bgv = pick(bgv, _K + j)

    idxs = idxsel_ref[...]
    R = jnp.where(idxs // _LANES == ri160, 1.0, 0.0)
    C = jnp.where(idxs % _LANES == li, 1.0, 0.0)
    dn = (((1,), (1,)), ((), ()))

    def gather_plane(p):
        t = lax.dot_general(C, p, dn, **_DOT)
        return jnp.sum(t * R, axis=1, keepdims=True)

    ex1 = gather_plane(planes_ref[0])
    ey1 = gather_plane(planes_ref[1])
    ex2 = gather_plane(planes_ref[2])
    ey2 = gather_plane(planes_ref[3])
    a = gather_plane(asg_ref[...])

    rois_ref[...] = jnp.where(li4 == 0, ex1,
                     jnp.where(li4 == 1, ey1,
                      jnp.where(li4 == 2, ex2, ey2)))

    gi64 = lax.broadcasted_iota(i32, (1, _G), 1)
    A = jnp.where(a[:_K].astype(i32) == gi64, 1.0, 0.0)
    dnr = (((1,), (0,)), ((), ()))
    glab = lax.dot_general(A, lab_ref[...], dnr, **_DOT)
    gbox = lax.dot_general(A, gt_ref[...], dnr, **_DOT)

    def gext(lane):
        return jnp.sum(jnp.where(li4 == lane, gbox, 0.0),
                       axis=1, keepdims=True)
    gx1, gy1, gx2, gy2 = gext(0), gext(1), gext(2), gext(3)

    fx1, fy1, fx2, fy2 = ex1[:_K], ey1[:_K], ex2[:_K], ey2[:_K]
    ex_w = fx2 - fx1 + 1.0
    ex_h = fy2 - fy1 + 1.0
    ex_cx = fx1 + 0.5 * ex_w
    ex_cy = fy1 + 0.5 * ex_h
    gt_w = gx2 - gx1 + 1.0
    gt_h = gy2 - gy1 + 1.0
    gt_cx = gx1 + 0.5 * gt_w
    gt_cy = gy1 + 0.5 * gt_h
    dx = (gt_cx - ex_cx) / ex_w
    dy = (gt_cy - ex_cy) / ex_h
    dw = jnp.log(gt_w / ex_w)
    dh = jnp.log(gt_h / ex_h)

    bbox_ref[pl.ds(0, _K), :] = jnp.where(li4 == 0, dx,
                                 jnp.where(li4 == 1, dy,
                                  jnp.where(li4 == 2, dw, dh)))
    bbox_ref[pl.ds(_K, _K), :] = jnp.zeros((_K, 4), f32)

    labels_ref[pl.ds(0, _K), :] = glab
    labels_ref[pl.ds(_K, _K), :] = jnp.where(li21 == 0,
                                             jnp.ones((_K, _C), f32), 0.0)


def kernel(proposals, bounding_boxes, labels):
    f32 = jnp.float32
    p = jnp.concatenate([proposals[0], bounding_boxes[0]], axis=0)
    pp = jnp.pad(p, ((0, _NPAD - _N), (0, 0)))
    planes = pp.T.reshape(4, _ROWS, _LANES)
    gt = bounding_boxes[0]
    lab = labels[0]

    out_shape = [jax.ShapeDtypeStruct((2 * _K, 4), f32),
                 jax.ShapeDtypeStruct((2 * _K, _C), f32),
                 jax.ShapeDtypeStruct((2 * _K, 4), f32)]
    rois, labels_out, bbox = pl.pallas_call(
        _proposal_target_kernel,
        out_shape=out_shape,
        in_specs=[
            pl.BlockSpec(memory_space=pltpu.SMEM),
            pl.BlockSpec(memory_space=pltpu.VMEM),
            pl.BlockSpec(memory_space=pltpu.VMEM),
            pl.BlockSpec(memory_space=pltpu.VMEM),
        ],
        out_specs=[pl.BlockSpec(memory_space=pltpu.VMEM)] * 3,
        scratch_shapes=[
            pltpu.VMEM((_ROWS, _LANES), f32),
            pltpu.VMEM((_ROWS, _LANES), f32),
            pltpu.VMEM((_ROWS, _LANES), f32),
            pltpu.VMEM((2 * _K, 1), jnp.int32),
        ],
    )(gt, planes, gt, lab)
    return (rois[None], labels_out[None], bbox[None])
